# baseline (speedup 1.0000x reference)
import functools

import jax
import jax.numpy as jnp
from jax import lax
from jax.experimental import pallas as pl
from jax.experimental.pallas import tpu as pltpu
from jax.experimental.pallas import tpu_sc as plsc

B = 16384
L = 50
D = 16
W_SHORT = 0.5
N_ITEMS = 1000000
N_USERS = 100000

NC = 2
NS = 16
NW = NC * NS

CH = 128
HB = B // 2
SEQ_CHUNKS = HB * L // CH
CPW = SEQ_CHUNKS // NW
NB = 3
SCH_PER_W = (HB // CH) // NW

_f32 = jnp.float32


def _transpose_chunk(wide_b, t_b, iota16):
    def d_step(d, carry):
        cols = jnp.zeros((16,), jnp.int32) + d
        for j0 in range(CH // 16):
            rows = iota16 + j0 * 16
            vals = plsc.load_gather(wide_b, [rows, cols])
            t_b[d, pl.ds(j0 * 16, 16)] = vals
        return carry

    lax.fori_loop(0, D, d_step, 0)


def _gather_body(seq_idx, pos_idx, neg_idx, usr_idx,
                 item_t, item2_t, user_t,
                 seq_out, pos_out, neg_out, pos2_out, neg2_out, usr_out,
                 idx_v, sidx_v, wide_v, t_v, gsem, wsem):
    w = lax.axis_index("s") * NC + lax.axis_index("c")
    iota16 = lax.iota(jnp.int32, 16)
    base_ch = w * CPW

    pltpu.sync_copy(seq_idx.at[w], idx_v)
    pltpu.sync_copy(pos_idx.at[w], sidx_v.at[0])
    pltpu.sync_copy(neg_idx.at[w], sidx_v.at[1])
    pltpu.sync_copy(usr_idx.at[w], sidx_v.at[2])

    def fire_seq(c, b):
        pltpu.async_copy(item_t.at[idx_v.at[c]], wide_v.at[b], gsem.at[b])

    for b in range(NB):
        fire_seq(b, b)

    def main_step(o, carry):
        for b in range(NB):
            c = o * NB + b

            @pl.when(c < CPW)
            def _handle(c=c, b=b):
                pltpu.make_async_copy(item_t.at[pl.ds(0, CH)],
                                      wide_v.at[b], gsem.at[b]).wait()

                @pl.when(c >= NB)
                def _drain_w():
                    pltpu.make_async_copy(t_v.at[b],
                                          seq_out.at[:, 0, pl.ds(0, CH)],
                                          wsem.at[b]).wait()

                cg = base_ch + c
                li = cg // (HB // CH)
                b0 = (cg % (HB // CH)) * CH
                _transpose_chunk(wide_v.at[b], t_v.at[b], iota16)
                pltpu.async_copy(t_v.at[b],
                                 seq_out.at[:, li, pl.ds(b0, CH)],
                                 wsem.at[b])

                @pl.when(c + NB < CPW)
                def _fire():
                    fire_seq(c + NB, b)

        return carry

    lax.fori_loop(0, (CPW + NB - 1) // NB, main_step, 0)

    for b in range(NB):
        pltpu.make_async_copy(t_v.at[b], seq_out.at[:, 0, pl.ds(0, CH)],
                              wsem.at[b]).wait()

    jobs = []
    for tab, si, outr in [(item_t, 0, pos_out), (item2_t, 0, pos2_out),
                          (item_t, 1, neg_out), (item2_t, 1, neg2_out),
                          (user_t, 2, usr_out)]:
        for r in range(SCH_PER_W):
            jobs.append((tab, si, r, outr))

    def fire_job(i, b):
        tab, si, r, _ = jobs[i]
        pltpu.async_copy(tab.at[sidx_v.at[si].at[r]], wide_v.at[b],
                         gsem.at[b])

    for b in range(NB):
        fire_job(b, b)
    for i, (tab, si, r, outr) in enumerate(jobs):
        b = i % NB
        pltpu.make_async_copy(item_t.at[pl.ds(0, CH)], wide_v.at[b],
                              gsem.at[b]).wait()
        if i >= NB:
            pltpu.make_async_copy(t_v.at[b], seq_out.at[:, 0, pl.ds(0, CH)],
                                  wsem.at[b]).wait()
        _transpose_chunk(wide_v.at[b], t_v.at[b], iota16)
        pltpu.async_copy(t_v.at[b],
                         outr.at[:, pl.ds(w * SCH_PER_W * CH + r * CH, CH)],
                         wsem.at[b])
        if i + NB < len(jobs):
            fire_job(i + NB, b)
    for b in range(NB):
        pltpu.make_async_copy(t_v.at[b], seq_out.at[:, 0, pl.ds(0, CH)],
                              wsem.at[b]).wait()


def _sc_gather(seq_idx, pos_idx, neg_idx, usr_idx, item_t, item2_t, user_t):
    mesh = plsc.VectorSubcoreMesh(core_axis_name="c", subcore_axis_name="s")
    out_type = [
        jax.ShapeDtypeStruct((D, L, HB), _f32),
        jax.ShapeDtypeStruct((D, HB), _f32),
        jax.ShapeDtypeStruct((D, HB), _f32),
        jax.ShapeDtypeStruct((D, HB), _f32),
        jax.ShapeDtypeStruct((D, HB), _f32),
        jax.ShapeDtypeStruct((D, HB), _f32),
    ]
    scratch = [
        pltpu.VMEM((CPW, CH), jnp.int32),
        pltpu.VMEM((3, SCH_PER_W, CH), jnp.int32),
        pltpu.VMEM((NB, CH, D), _f32),
        pltpu.VMEM((NB, D, CH), _f32),
        pltpu.SemaphoreType.DMA((NB,)),
        pltpu.SemaphoreType.DMA((NB,)),
    ]
    fn = pl.kernel(_gather_body, mesh=mesh, out_type=out_type,
                   scratch_types=scratch,
                   compiler_params=pltpu.CompilerParams(
                       use_tc_tiling_on_sc=False,
                       needs_layout_passes=False))
    return fn(seq_idx, pos_idx, neg_idx, usr_idx, item_t, item2_t, user_t)


def _attn_body(seq_ref, midx_ref, usr_ref, pos_ref, neg_ref, pos2_ref,
               neg2_ref, w_ref, pos_s_ref, neg_s_ref, q_ref):
    bb = seq_ref.shape[2]
    valid = midx_ref[...] != 0
    neg_big = jnp.float32(-2.0 ** 32 + 1.0)

    for e in range(D):
        acc = seq_ref[0] * w_ref[0, e]
        for d in range(1, D):
            acc = acc + seq_ref[d] * w_ref[d, e]
        q_ref[e] = jnp.maximum(acc, 0.0)

    GL = 5
    wcol = jnp.zeros((L, bb), _f32)
    for l0 in range(0, L, GL):
        accs = [None] * GL
        for e in range(D):
            qe = q_ref[e]
            for g in range(GL):
                t = qe * qe[l0 + g][None, :]
                accs[g] = t if e == 0 else accs[g] + t
        wg = None
        for g in range(GL):
            sc = jnp.where(valid, accs[g] * 0.25, neg_big)
            mx = jnp.max(sc, axis=0, keepdims=True)
            ee = jnp.exp(sc - mx)
            dn = jnp.sum(ee, axis=0, keepdims=True)
            t = ee * (1.0 / dn)
            wg = t if wg is None else wg + t
        wcol = wcol + wg

    srows = []
    for d in range(D):
        srows.append(jnp.sum(seq_ref[d] * wcol, axis=0, keepdims=True))
    short = jnp.concatenate(srows, axis=0) * (1.0 / L)

    u = usr_ref[...]
    pos_s = (W_SHORT * jnp.sum(u * pos2_ref[...], axis=0, keepdims=True)
             + (1.0 - W_SHORT) * jnp.sum(short * pos_ref[...], axis=0,
                                         keepdims=True))
    neg_s = (W_SHORT * jnp.sum(u * neg2_ref[...], axis=0, keepdims=True)
             + (1.0 - W_SHORT) * jnp.sum(short * neg_ref[...], axis=0,
                                         keepdims=True))
    pos_s_ref[...] = pos_s
    neg_s_ref[...] = neg_s


def _tc_attention(seq_t, mask_idx_t, usr_t, pos_t, neg_t, pos2_t, neg2_t,
                  W_attn, bb=128):
    grid = (HB // bb,)
    vec = pl.BlockSpec((D, bb), lambda i: (0, i))
    outs = pl.pallas_call(
        _attn_body,
        grid=grid,
        in_specs=[
            pl.BlockSpec((D, L, bb), lambda i: (0, 0, i)),
            pl.BlockSpec((L, bb), lambda i: (0, i)),
            vec, vec, vec, vec, vec,
            pl.BlockSpec(memory_space=pltpu.SMEM),
        ],
        out_specs=[pl.BlockSpec((1, bb), lambda i: (0, i)),
                   pl.BlockSpec((1, bb), lambda i: (0, i))],
        out_shape=[jax.ShapeDtypeStruct((1, HB), _f32),
                   jax.ShapeDtypeStruct((1, HB), _f32)],
        scratch_shapes=[pltpu.VMEM((D, L, bb), _f32)],
    )(seq_t, mask_idx_t, usr_t, pos_t, neg_t, pos2_t, neg2_t, W_attn)
    return outs


def kernel(user_inputs, seq_inputs, pos_inputs, neg_inputs, user_table,
           item_table, item2_table, W_attn):
    item_r = item_table
    item2_r = item2_table
    user_r = user_table
    seq_T = seq_inputs.T
    outs = []
    for h in range(2):
        sl = slice(h * HB, (h + 1) * HB)
        seq_idx_t = seq_T[:, sl].reshape(NW, CPW, CH)
        pos_idx = pos_inputs[sl].reshape(NW, SCH_PER_W, CH)
        neg_idx = neg_inputs[sl].reshape(NW, SCH_PER_W, CH)
        usr_idx = user_inputs[sl].reshape(NW, SCH_PER_W, CH)
        seq_t, pos_t, neg_t, pos2_t, neg2_t, usr_t = _sc_gather(
            seq_idx_t, pos_idx, neg_idx, usr_idx, item_r, item2_r, user_r)
        outs.append(_tc_attention(seq_t, seq_T[:, sl], usr_t, pos_t,
                                  neg_t, pos2_t, neg2_t, W_attn))
    pos_s = jnp.concatenate([outs[0][0], outs[1][0]], axis=1)
    neg_s = jnp.concatenate([outs[0][1], outs[1][1]], axis=1)
    return (pos_s.reshape(B, 1), neg_s.reshape(B, 1))


# speedup vs baseline: 1.0946x; 1.0946x over previous
import functools

import jax
import jax.numpy as jnp
from jax import lax
from jax.experimental import pallas as pl
from jax.experimental.pallas import tpu as pltpu
from jax.experimental.pallas import tpu_sc as plsc

B = 16384
L = 50
D = 16
W_SHORT = 0.5
N_ITEMS = 1000000
N_USERS = 100000

NC = 2
NS = 16
NW = NC * NS

CH = 128
HB = B // 2
SEQ_CHUNKS = HB * L // CH
CPW = SEQ_CHUNKS // NW
NB = 3
SCH_PER_W = (HB // CH) // NW

_f32 = jnp.float32


def _transpose_chunk(wide_b, off_row, t_b, iota16):
    def j_step(j0, carry):
        offs = off_row[pl.ds(j0 * 16, 16)]
        rows = iota16 + j0 * 16
        for d in range(D):
            vals = plsc.load_gather(wide_b, [rows, offs + d])
            t_b[d, pl.ds(j0 * 16, 16)] = vals
        return carry

    lax.fori_loop(0, CH // 16, j_step, 0)


def _gather_body(seq_idx, pos_idx, neg_idx, usr_idx,
                 item_t, item2_t, user_t,
                 seq_out, pos_out, neg_out, pos2_out, neg2_out, usr_out,
                 idx8_v, off_v, sidx8_v, soff_v, wide_v, t_v, gsem, wsem):
    w = lax.axis_index("s") * NC + lax.axis_index("c")
    iota16 = lax.iota(jnp.int32, 16)
    base_ch = w * CPW

    pltpu.sync_copy(seq_idx.at[w], idx8_v)
    pltpu.sync_copy(pos_idx.at[w], sidx8_v.at[0])
    pltpu.sync_copy(neg_idx.at[w], sidx8_v.at[1])
    pltpu.sync_copy(usr_idx.at[w], sidx8_v.at[2])

    def split_row(r, carry):
        for j0 in range(CH // 16):
            sl = pl.ds(j0 * 16, 16)
            raw = idx8_v[r, sl]
            off_v[r, sl] = (raw & 7) << 4
            idx8_v[r, sl] = raw >> 3
        return carry

    lax.fori_loop(0, CPW, split_row, 0)

    for si in range(3):
        for r in range(SCH_PER_W):
            for j0 in range(CH // 16):
                sl = pl.ds(j0 * 16, 16)
                raw = sidx8_v[si, r, sl]
                soff_v[si, r, sl] = (raw & 7) << 4
                sidx8_v[si, r, sl] = raw >> 3

    def fire_seq(c, b):
        pltpu.async_copy(item_t.at[idx8_v.at[c]], wide_v.at[b], gsem.at[b])

    for b in range(NB):
        fire_seq(b, b)

    def main_step(o, carry):
        for b in range(NB):
            c = o * NB + b

            @pl.when(c < CPW)
            def _handle(c=c, b=b):
                pltpu.make_async_copy(item_t.at[pl.ds(0, CH)],
                                      wide_v.at[b], gsem.at[b]).wait()

                @pl.when(c >= NB)
                def _drain_w():
                    pltpu.make_async_copy(t_v.at[b],
                                          seq_out.at[:, 0, pl.ds(0, CH)],
                                          wsem.at[b]).wait()

                cg = base_ch + c
                li = cg // (HB // CH)
                b0 = (cg % (HB // CH)) * CH
                _transpose_chunk(wide_v.at[b], off_v.at[c], t_v.at[b],
                                 iota16)
                pltpu.async_copy(t_v.at[b],
                                 seq_out.at[:, li, pl.ds(b0, CH)],
                                 wsem.at[b])

                @pl.when(c + NB < CPW)
                def _fire():
                    fire_seq(c + NB, b)

        return carry

    lax.fori_loop(0, (CPW + NB - 1) // NB, main_step, 0)

    for b in range(NB):
        pltpu.make_async_copy(t_v.at[b], seq_out.at[:, 0, pl.ds(0, CH)],
                              wsem.at[b]).wait()

    jobs = []
    for tab, si, outr in [(item_t, 0, pos_out), (item2_t, 0, pos2_out),
                          (item_t, 1, neg_out), (item2_t, 1, neg2_out),
                          (user_t, 2, usr_out)]:
        for r in range(SCH_PER_W):
            jobs.append((tab, si, r, outr))

    def fire_job(i, b):
        tab, si, r, _ = jobs[i]
        pltpu.async_copy(tab.at[sidx8_v.at[si].at[r]], wide_v.at[b],
                         gsem.at[b])

    for b in range(NB):
        fire_job(b, b)
    for i, (tab, si, r, outr) in enumerate(jobs):
        b = i % NB
        pltpu.make_async_copy(item_t.at[pl.ds(0, CH)], wide_v.at[b],
                              gsem.at[b]).wait()
        if i >= NB:
            pltpu.make_async_copy(t_v.at[b], seq_out.at[:, 0, pl.ds(0, CH)],
                                  wsem.at[b]).wait()
        _transpose_chunk(wide_v.at[b], soff_v.at[si].at[r], t_v.at[b],
                         iota16)
        pltpu.async_copy(t_v.at[b],
                         outr.at[:, pl.ds(w * SCH_PER_W * CH + r * CH, CH)],
                         wsem.at[b])
        if i + NB < len(jobs):
            fire_job(i + NB, b)
    for b in range(NB):
        pltpu.make_async_copy(t_v.at[b], seq_out.at[:, 0, pl.ds(0, CH)],
                              wsem.at[b]).wait()


def _sc_gather(seq_idx, pos_idx, neg_idx, usr_idx, item_t, item2_t, user_t):
    mesh = plsc.VectorSubcoreMesh(core_axis_name="c", subcore_axis_name="s")
    out_type = [
        jax.ShapeDtypeStruct((D, L, HB), _f32),
        jax.ShapeDtypeStruct((D, HB), _f32),
        jax.ShapeDtypeStruct((D, HB), _f32),
        jax.ShapeDtypeStruct((D, HB), _f32),
        jax.ShapeDtypeStruct((D, HB), _f32),
        jax.ShapeDtypeStruct((D, HB), _f32),
    ]
    scratch = [
        pltpu.VMEM((CPW, CH), jnp.int32),
        pltpu.VMEM((CPW, CH), jnp.int32),
        pltpu.VMEM((3, SCH_PER_W, CH), jnp.int32),
        pltpu.VMEM((3, SCH_PER_W, CH), jnp.int32),
        pltpu.VMEM((NB, CH, CH), _f32),
        pltpu.VMEM((NB, D, CH), _f32),
        pltpu.SemaphoreType.DMA((NB,)),
        pltpu.SemaphoreType.DMA((NB,)),
    ]
    fn = pl.kernel(_gather_body, mesh=mesh, out_type=out_type,
                   scratch_types=scratch,
                   compiler_params=pltpu.CompilerParams(
                       use_tc_tiling_on_sc=True,
                       needs_layout_passes=False))
    return fn(seq_idx, pos_idx, neg_idx, usr_idx, item_t, item2_t, user_t)


def _attn_body(seq_ref, midx_ref, usr_ref, pos_ref, neg_ref, pos2_ref,
               neg2_ref, w_ref, pos_s_ref, neg_s_ref, q_ref):
    bb = seq_ref.shape[2]
    valid = midx_ref[...] != 0
    neg_big = jnp.float32(-2.0 ** 32 + 1.0)

    for e in range(D):
        acc = seq_ref[0] * w_ref[0, e]
        for d in range(1, D):
            acc = acc + seq_ref[d] * w_ref[d, e]
        q_ref[e] = jnp.maximum(acc, 0.0)

    GL = 5
    wcol = jnp.zeros((L, bb), _f32)
    for l0 in range(0, L, GL):
        accs = [None] * GL
        for e in range(D):
            qe = q_ref[e]
            for g in range(GL):
                t = qe * qe[l0 + g][None, :]
                accs[g] = t if e == 0 else accs[g] + t
        wg = None
        for g in range(GL):
            sc = jnp.where(valid, accs[g] * 0.25, neg_big)
            mx = jnp.max(sc, axis=0, keepdims=True)
            ee = jnp.exp(sc - mx)
            dn = jnp.sum(ee, axis=0, keepdims=True)
            t = ee * (1.0 / dn)
            wg = t if wg is None else wg + t
        wcol = wcol + wg

    srows = []
    for d in range(D):
        srows.append(jnp.sum(seq_ref[d] * wcol, axis=0, keepdims=True))
    short = jnp.concatenate(srows, axis=0) * (1.0 / L)

    u = usr_ref[...]
    pos_s = (W_SHORT * jnp.sum(u * pos2_ref[...], axis=0, keepdims=True)
             + (1.0 - W_SHORT) * jnp.sum(short * pos_ref[...], axis=0,
                                         keepdims=True))
    neg_s = (W_SHORT * jnp.sum(u * neg2_ref[...], axis=0, keepdims=True)
             + (1.0 - W_SHORT) * jnp.sum(short * neg_ref[...], axis=0,
                                         keepdims=True))
    pos_s_ref[...] = pos_s
    neg_s_ref[...] = neg_s


def _tc_attention(seq_t, mask_idx_t, usr_t, pos_t, neg_t, pos2_t, neg2_t,
                  W_attn, bb=128):
    grid = (HB // bb,)
    vec = pl.BlockSpec((D, bb), lambda i: (0, i))
    outs = pl.pallas_call(
        _attn_body,
        grid=grid,
        in_specs=[
            pl.BlockSpec((D, L, bb), lambda i: (0, 0, i)),
            pl.BlockSpec((L, bb), lambda i: (0, i)),
            vec, vec, vec, vec, vec,
            pl.BlockSpec(memory_space=pltpu.SMEM),
        ],
        out_specs=[pl.BlockSpec((1, bb), lambda i: (0, i)),
                   pl.BlockSpec((1, bb), lambda i: (0, i))],
        out_shape=[jax.ShapeDtypeStruct((1, HB), _f32),
                   jax.ShapeDtypeStruct((1, HB), _f32)],
        scratch_shapes=[pltpu.VMEM((D, L, bb), _f32)],
    )(seq_t, mask_idx_t, usr_t, pos_t, neg_t, pos2_t, neg2_t, W_attn)
    return outs


def kernel(user_inputs, seq_inputs, pos_inputs, neg_inputs, user_table,
           item_table, item2_table, W_attn):
    item_r = item_table.reshape(N_ITEMS // 8, 128)
    item2_r = item2_table.reshape(N_ITEMS // 8, 128)
    user_r = user_table.reshape(N_USERS // 8, 128)
    seq_T = seq_inputs.T
    outs = []
    for h in range(2):
        sl = slice(h * HB, (h + 1) * HB)
        seq_idx_t = seq_T[:, sl].reshape(NW, CPW, CH)
        pos_idx = pos_inputs[sl].reshape(NW, SCH_PER_W, CH)
        neg_idx = neg_inputs[sl].reshape(NW, SCH_PER_W, CH)
        usr_idx = user_inputs[sl].reshape(NW, SCH_PER_W, CH)
        seq_t, pos_t, neg_t, pos2_t, neg2_t, usr_t = _sc_gather(
            seq_idx_t, pos_idx, neg_idx, usr_idx, item_r, item2_r, user_r)
        outs.append(_tc_attention(seq_t, seq_T[:, sl], usr_t, pos_t,
                                  neg_t, pos2_t, neg2_t, W_attn))
    pos_s = jnp.concatenate([outs[0][0], outs[1][0]], axis=1)
    neg_s = jnp.concatenate([outs[0][1], outs[1][1]], axis=1)
    return (pos_s.reshape(B, 1), neg_s.reshape(B, 1))


# speedup vs baseline: 1.1459x; 1.0469x over previous
import functools

import jax
import jax.numpy as jnp
from jax import lax
from jax.experimental import pallas as pl
from jax.experimental.pallas import tpu as pltpu
from jax.experimental.pallas import tpu_sc as plsc

B = 16384
L = 50
D = 16
W_SHORT = 0.5
N_ITEMS = 1000000
N_USERS = 100000

NC = 2
NS = 16
NW = NC * NS

CH = 128
HB = B // 4
SEQ_CHUNKS = HB * L // CH
CPW = SEQ_CHUNKS // NW
NB = 3
SCH_PER_W = (HB // CH) // NW

_f32 = jnp.float32


def _transpose_chunk(wide_b, off_row, t_b, iota16):
    def j_step(j0, carry):
        offs = off_row[pl.ds(j0 * 16, 16)]
        rows = iota16 + j0 * 16
        for d in range(D):
            vals = plsc.load_gather(wide_b, [rows, offs + d])
            t_b[d, pl.ds(j0 * 16, 16)] = vals
        return carry

    lax.fori_loop(0, CH // 16, j_step, 0)


def _gather_body(seq_idx, pos_idx, neg_idx, usr_idx,
                 item_t, item2_t, user_t,
                 seq_out, pos_out, neg_out, pos2_out, neg2_out, usr_out,
                 idx8_v, off_v, sidx8_v, soff_v, wide_v, t_v, gsem, wsem):
    w = lax.axis_index("s") * NC + lax.axis_index("c")
    iota16 = lax.iota(jnp.int32, 16)
    base_ch = w * CPW

    pltpu.sync_copy(seq_idx.at[w], idx8_v)
    pltpu.sync_copy(pos_idx.at[w], sidx8_v.at[0])
    pltpu.sync_copy(neg_idx.at[w], sidx8_v.at[1])
    pltpu.sync_copy(usr_idx.at[w], sidx8_v.at[2])

    def split_row(r, carry):
        for j0 in range(CH // 16):
            sl = pl.ds(j0 * 16, 16)
            raw = idx8_v[r, sl]
            off_v[r, sl] = (raw & 7) << 4
            idx8_v[r, sl] = raw >> 3
        return carry

    lax.fori_loop(0, CPW, split_row, 0)

    for si in range(3):
        for r in range(SCH_PER_W):
            for j0 in range(CH // 16):
                sl = pl.ds(j0 * 16, 16)
                raw = sidx8_v[si, r, sl]
                soff_v[si, r, sl] = (raw & 7) << 4
                sidx8_v[si, r, sl] = raw >> 3

    def fire_seq(c, b):
        pltpu.async_copy(item_t.at[idx8_v.at[c]], wide_v.at[b], gsem.at[b])

    for b in range(NB):
        fire_seq(b, b)

    def main_step(o, carry):
        for b in range(NB):
            c = o * NB + b

            @pl.when(c < CPW)
            def _handle(c=c, b=b):
                pltpu.make_async_copy(item_t.at[pl.ds(0, CH)],
                                      wide_v.at[b], gsem.at[b]).wait()

                @pl.when(c >= NB)
                def _drain_w():
                    pltpu.make_async_copy(t_v.at[b],
                                          seq_out.at[:, 0, pl.ds(0, CH)],
                                          wsem.at[b]).wait()

                cg = base_ch + c
                li = cg // (HB // CH)
                b0 = (cg % (HB // CH)) * CH
                _transpose_chunk(wide_v.at[b], off_v.at[c], t_v.at[b],
                                 iota16)
                pltpu.async_copy(t_v.at[b],
                                 seq_out.at[:, li, pl.ds(b0, CH)],
                                 wsem.at[b])

                @pl.when(c + NB < CPW)
                def _fire():
                    fire_seq(c + NB, b)

        return carry

    lax.fori_loop(0, (CPW + NB - 1) // NB, main_step, 0)

    for b in range(NB):
        pltpu.make_async_copy(t_v.at[b], seq_out.at[:, 0, pl.ds(0, CH)],
                              wsem.at[b]).wait()

    jobs = []
    for tab, si, outr in [(item_t, 0, pos_out), (item2_t, 0, pos2_out),
                          (item_t, 1, neg_out), (item2_t, 1, neg2_out),
                          (user_t, 2, usr_out)]:
        for r in range(SCH_PER_W):
            jobs.append((tab, si, r, outr))

    def fire_job(i, b):
        tab, si, r, _ = jobs[i]
        pltpu.async_copy(tab.at[sidx8_v.at[si].at[r]], wide_v.at[b],
                         gsem.at[b])

    for b in range(NB):
        fire_job(b, b)
    for i, (tab, si, r, outr) in enumerate(jobs):
        b = i % NB
        pltpu.make_async_copy(item_t.at[pl.ds(0, CH)], wide_v.at[b],
                              gsem.at[b]).wait()
        if i >= NB:
            pltpu.make_async_copy(t_v.at[b], seq_out.at[:, 0, pl.ds(0, CH)],
                                  wsem.at[b]).wait()
        _transpose_chunk(wide_v.at[b], soff_v.at[si].at[r], t_v.at[b],
                         iota16)
        pltpu.async_copy(t_v.at[b],
                         outr.at[:, pl.ds(w * SCH_PER_W * CH + r * CH, CH)],
                         wsem.at[b])
        if i + NB < len(jobs):
            fire_job(i + NB, b)
    for b in range(NB):
        pltpu.make_async_copy(t_v.at[b], seq_out.at[:, 0, pl.ds(0, CH)],
                              wsem.at[b]).wait()


def _sc_gather(seq_idx, pos_idx, neg_idx, usr_idx, item_t, item2_t, user_t):
    mesh = plsc.VectorSubcoreMesh(core_axis_name="c", subcore_axis_name="s")
    out_type = [
        jax.ShapeDtypeStruct((D, L, HB), _f32),
        jax.ShapeDtypeStruct((D, HB), _f32),
        jax.ShapeDtypeStruct((D, HB), _f32),
        jax.ShapeDtypeStruct((D, HB), _f32),
        jax.ShapeDtypeStruct((D, HB), _f32),
        jax.ShapeDtypeStruct((D, HB), _f32),
    ]
    scratch = [
        pltpu.VMEM((CPW, CH), jnp.int32),
        pltpu.VMEM((CPW, CH), jnp.int32),
        pltpu.VMEM((3, SCH_PER_W, CH), jnp.int32),
        pltpu.VMEM((3, SCH_PER_W, CH), jnp.int32),
        pltpu.VMEM((NB, CH, CH), _f32),
        pltpu.VMEM((NB, D, CH), _f32),
        pltpu.SemaphoreType.DMA((NB,)),
        pltpu.SemaphoreType.DMA((NB,)),
    ]
    fn = pl.kernel(_gather_body, mesh=mesh, out_type=out_type,
                   scratch_types=scratch,
                   compiler_params=pltpu.CompilerParams(
                       use_tc_tiling_on_sc=True,
                       needs_layout_passes=False))
    return fn(seq_idx, pos_idx, neg_idx, usr_idx, item_t, item2_t, user_t)


def _attn_body(seq_ref, midx_ref, usr_ref, pos_ref, neg_ref, pos2_ref,
               neg2_ref, w_ref, pos_s_ref, neg_s_ref, q_ref):
    bb = seq_ref.shape[2]
    valid = midx_ref[...] != 0
    neg_big = jnp.float32(-2.0 ** 32 + 1.0)

    for e in range(D):
        acc = seq_ref[0] * w_ref[0, e]
        for d in range(1, D):
            acc = acc + seq_ref[d] * w_ref[d, e]
        q_ref[e] = jnp.maximum(acc, 0.0)

    GL = 5
    wcol = jnp.zeros((L, bb), _f32)
    for l0 in range(0, L, GL):
        accs = [None] * GL
        for e in range(D):
            qe = q_ref[e]
            for g in range(GL):
                t = qe * qe[l0 + g][None, :]
                accs[g] = t if e == 0 else accs[g] + t
        wg = None
        for g in range(GL):
            sc = jnp.where(valid, accs[g] * 0.25, neg_big)
            mx = jnp.max(sc, axis=0, keepdims=True)
            ee = jnp.exp(sc - mx)
            dn = jnp.sum(ee, axis=0, keepdims=True)
            t = ee * (1.0 / dn)
            wg = t if wg is None else wg + t
        wcol = wcol + wg

    srows = []
    for d in range(D):
        srows.append(jnp.sum(seq_ref[d] * wcol, axis=0, keepdims=True))
    short = jnp.concatenate(srows, axis=0) * (1.0 / L)

    u = usr_ref[...]
    pos_s = (W_SHORT * jnp.sum(u * pos2_ref[...], axis=0, keepdims=True)
             + (1.0 - W_SHORT) * jnp.sum(short * pos_ref[...], axis=0,
                                         keepdims=True))
    neg_s = (W_SHORT * jnp.sum(u * neg2_ref[...], axis=0, keepdims=True)
             + (1.0 - W_SHORT) * jnp.sum(short * neg_ref[...], axis=0,
                                         keepdims=True))
    pos_s_ref[...] = pos_s
    neg_s_ref[...] = neg_s


def _tc_attention(seq_t, mask_idx_t, usr_t, pos_t, neg_t, pos2_t, neg2_t,
                  W_attn, bb=128):
    grid = (HB // bb,)
    vec = pl.BlockSpec((D, bb), lambda i: (0, i))
    outs = pl.pallas_call(
        _attn_body,
        grid=grid,
        in_specs=[
            pl.BlockSpec((D, L, bb), lambda i: (0, 0, i)),
            pl.BlockSpec((L, bb), lambda i: (0, i)),
            vec, vec, vec, vec, vec,
            pl.BlockSpec(memory_space=pltpu.SMEM),
        ],
        out_specs=[pl.BlockSpec((1, bb), lambda i: (0, i)),
                   pl.BlockSpec((1, bb), lambda i: (0, i))],
        out_shape=[jax.ShapeDtypeStruct((1, HB), _f32),
                   jax.ShapeDtypeStruct((1, HB), _f32)],
        scratch_shapes=[pltpu.VMEM((D, L, bb), _f32)],
    )(seq_t, mask_idx_t, usr_t, pos_t, neg_t, pos2_t, neg2_t, W_attn)
    return outs


def kernel(user_inputs, seq_inputs, pos_inputs, neg_inputs, user_table,
           item_table, item2_table, W_attn):
    item_r = item_table.reshape(N_ITEMS // 8, 128)
    item2_r = item2_table.reshape(N_ITEMS // 8, 128)
    user_r = user_table.reshape(N_USERS // 8, 128)
    seq_T = seq_inputs.T
    outs = []
    for h in range(4):
        sl = slice(h * HB, (h + 1) * HB)
        seq_idx_t = seq_T[:, sl].reshape(NW, CPW, CH)
        pos_idx = pos_inputs[sl].reshape(NW, SCH_PER_W, CH)
        neg_idx = neg_inputs[sl].reshape(NW, SCH_PER_W, CH)
        usr_idx = user_inputs[sl].reshape(NW, SCH_PER_W, CH)
        seq_t, pos_t, neg_t, pos2_t, neg2_t, usr_t = _sc_gather(
            seq_idx_t, pos_idx, neg_idx, usr_idx, item_r, item2_r, user_r)
        outs.append(_tc_attention(seq_t, seq_T[:, sl], usr_t, pos_t,
                                  neg_t, pos2_t, neg2_t, W_attn))
    pos_s = jnp.concatenate([o[0] for o in outs], axis=1)
    neg_s = jnp.concatenate([o[1] for o in outs], axis=1)
    return (pos_s.reshape(B, 1), neg_s.reshape(B, 1))
